# baseline (speedup 1.0000x reference)
import functools

import jax
import jax.numpy as jnp
from jax import lax
from jax.experimental import pallas as pl
from jax.experimental.pallas import tpu as pltpu
from jax.experimental.pallas import tpu_sc as plsc

N = 10000
M = 10000
F = 128
E = 320000

NC = 2
NS = 16
NW = NC * NS
L = 16

NP = 10240
HN = NP // NC
EW = E // NW
K = 80
EWP = K * 128
HTS = HN // NS

CH = 128
PCAP = K * 128 + 4 * CH

_BIG_NEG = -1e30
_IDX_BITS = 14
_IDX_MASK = (1 << _IDX_BITS) - 1


def _mm_scal_body(x_ref, w_ref, a_ref, h_ref, st_ref):
    xb = x_ref[...]
    hb = jnp.dot(xb, w_ref[...], preferred_element_type=jnp.float32)
    h_ref[...] = hb
    st_ref[...] = lax.dot_general(a_ref[...], hb, (((0,), (1,)), ((), ())),
                                  preferred_element_type=jnp.float32)


def _relu_mm_scal_body(f_ref, w_ref, a_ref, u_ref, st_ref):
    fb = jnp.maximum(f_ref[...], 0.0)
    ub = jnp.dot(fb, w_ref[...], preferred_element_type=jnp.float32)
    u_ref[...] = ub
    st_ref[...] = lax.dot_general(a_ref[...], ub, (((0,), (1,)), ((), ())),
                                  preferred_element_type=jnp.float32)


def _matmul_scal(x, w, acols, relu_input=False):
    blk = 1024
    body = _relu_mm_scal_body if relu_input else _mm_scal_body
    return pl.pallas_call(
        body,
        grid=(NP // blk,),
        in_specs=[
            pl.BlockSpec((blk, F), lambda i: (i, 0)),
            pl.BlockSpec((F, F), lambda i: (0, 0)),
            pl.BlockSpec((F, 8), lambda i: (0, 0)),
        ],
        out_specs=[
            pl.BlockSpec((blk, F), lambda i: (i, 0)),
            pl.BlockSpec((8, blk), lambda i: (0, i)),
        ],
        out_shape=[
            jax.ShapeDtypeStruct((NP, F), jnp.float32),
            jax.ShapeDtypeStruct((8, NP), jnp.float32),
        ],
    )(x, w, acols)


def _elu_body(o_ref, out_ref):
    o = o_ref[...]
    out_ref[...] = jnp.where(o > 0, o, jnp.exp(o) - 1.0)


def _elu(o_full):
    blk = 1000
    return pl.pallas_call(
        _elu_body,
        grid=(N // blk,),
        in_specs=[pl.BlockSpec((blk, F), lambda i: (i, 0))],
        out_specs=pl.BlockSpec((blk, F), lambda i: (i, 0)),
        out_shape=jax.ShapeDtypeStruct((N, F), jnp.float32),
    )(o_full)


_MESH = plsc.VectorSubcoreMesh(core_axis_name="c", subcore_axis_name="s",
                               num_cores=NC, num_subcores=NS)


def _fix_table_pads(tab_v):
    for t in range((NP - N) // L):
        tab_v[pl.ds(N + L * t, L)] = jnp.full((L,), _BIG_NEG, jnp.float32)


def _score_pass_body(two_tables, t1_row, t2_row,
                     t1_hbm, t2_hbm, pk_hbm, e_hbm,
                     t1_v, t2_v, pk_v, ev):
    c = lax.axis_index("c")
    s = lax.axis_index("s")
    wid = s * NC + c

    pltpu.sync_copy(t1_hbm.at[t1_row], t1_v)
    pltpu.sync_copy(pk_hbm.at[wid], pk_v)
    if two_tables:
        pltpu.sync_copy(t2_hbm.at[t2_row], t2_v)
    _fix_table_pads(t1_v)

    def compute(j, _):
        for cc in range(8):
            sl = pl.ds(L * cc, L)
            p16 = pk_v[j, sl]
            lo = p16 & _IDX_MASK
            hi = lax.shift_right_logical(p16, _IDX_BITS)
            raw = plsc.load_gather(t1_v, [lo])
            if two_tables:
                raw = raw + plsc.load_gather(t2_v, [hi])
            raw = jnp.where(raw > 0, raw, 0.2 * raw)
            ev[j, sl] = jnp.exp(raw)
        return 0

    lax.fori_loop(0, K, compute, 0)
    pltpu.sync_copy(ev, e_hbm.at[wid])


def _make_score_pass(two_tables, t1_row, t2_row):
    body = functools.partial(_score_pass_body, two_tables, t1_row, t2_row)
    return pl.kernel(
        body,
        out_type=[jax.ShapeDtypeStruct((NW, K, 128), jnp.float32)],
        mesh=_MESH,
        compiler_params=pltpu.CompilerParams(needs_layout_passes=False),
        scratch_types=[
            pltpu.VMEM((NP,), jnp.float32),
            pltpu.VMEM((NP,), jnp.float32),
            pltpu.VMEM((K, 128), jnp.int32),
            pltpu.VMEM((K, 128), jnp.float32),
        ],
    )


def _row_pass_body(gather_low,
                   rows_hbm, e_hbm, pk_hbm, acc_hbm,
                   pk_v, ev, pend_g, pend_l, pend_e,
                   lsx_a, lsx_b, evrow_a, evrow_b, rd_v, rows_a, rows_b,
                   acc_sh, den_sh, gsem_a, gsem_b, ssem_a, ssem_b):
    c = lax.axis_index("c")
    s = lax.axis_index("s")
    base = c * HN

    def zrow(r, _):
        for cc in range(8):
            rows_a[r, pl.ds(L * cc, L)] = jnp.zeros((L,), jnp.float32)
        return 0

    lax.fori_loop(0, CH, zrow, 0)
    for b in range(HTS // CH):
        pltpu.sync_copy(rows_a, acc_sh.at[pl.ds(HTS * s + CH * b, CH)])
    pltpu.sync_copy(rows_a.at[pl.ds(0, HTS % CH)],
                    acc_sh.at[pl.ds(HTS * s + (HTS // CH) * CH, HTS % CH)])
    for v in range(HTS // L):
        rd_v[pl.ds(L * v, L)] = jnp.zeros((L,), jnp.float32)
    pltpu.sync_copy(rd_v, den_sh.at[pl.ds(HTS * s, HTS)])
    plsc.subcore_barrier()

    def fire(j, buf, sem):
        pltpu.async_copy(rows_hbm.at[pend_g.at[pl.ds(CH * j, CH)]],
                         buf, sem)

    def wait(j, buf, sem):
        pltpu.make_async_copy(rows_hbm.at[pend_g.at[pl.ds(CH * j, CH)]],
                              buf, sem).wait()

    def process(j, buf, lsx, evrow):
        for cc in range(CH // L):
            sl = pl.ds(L * cc, L)
            p = pl.ds(CH * j + L * cc, L)
            lsx[0, sl] = pend_l[p]
            evrow[0, sl] = pend_e[p]

        def scale(r, _):
            w = plsc.load_gather(pend_e, [jnp.full((L,), CH * j + r,
                                                   jnp.int32)])
            for cc in range(8):
                sl = pl.ds(L * cc, L)
                buf[r, sl] = buf[r, sl] * w
            return 0

        lax.fori_loop(0, CH, scale, 0)

    def scat_fire(buf, lsx, evrow, sem):
        pltpu.async_copy(buf, acc_sh.at[lsx.at[0]], sem, add=True)
        pltpu.async_copy(evrow.at[0], den_sh.at[lsx.at[0]], sem, add=True)

    def scat_wait(buf, lsx, evrow, sem):
        pltpu.make_async_copy(buf, acc_sh.at[lsx.at[0]], sem).wait()
        pltpu.make_async_copy(evrow.at[0], den_sh.at[lsx.at[0]], sem).wait()

    for wsub in range(NC):
        wid = s * NC + wsub
        pltpu.sync_copy(pk_hbm.at[wid], pk_v)
        pltpu.sync_copy(e_hbm.at[wid], ev)

        def compact(j, cnt):
            for cc in range(8):
                sl = pl.ds(L * cc, L)
                p16 = pk_v[j, sl]
                lo = p16 & _IDX_MASK
                hi = lax.shift_right_logical(p16, _IDX_BITS)
                if gather_low:
                    g16, s16 = lo, hi
                else:
                    g16, s16 = hi, lo
                l16 = s16 - base
                msk = (l16 >= 0) & (l16 < HN)
                mi = msk.astype(jnp.int32)
                pos = cnt + plsc.cumsum(mi) - 1
                plsc.store_scatter(pend_g, [pos], g16, mask=msk)
                plsc.store_scatter(pend_l, [pos], l16, mask=msk)
                plsc.store_scatter(pend_e, [pos], ev[j, sl], mask=msk)
                cnt = cnt + jnp.sum(mi)
            return cnt

        cnt = lax.fori_loop(0, K, compact, jnp.int32(0))

        iota16 = lax.iota(jnp.int32, L)
        for t in range(4 * CH // L):
            pos = cnt + iota16 + L * t
            plsc.store_scatter(pend_g, [pos], jnp.zeros((L,), jnp.int32))
            plsc.store_scatter(pend_l, [pos], jnp.zeros((L,), jnp.int32))
            plsc.store_scatter(pend_e, [pos], jnp.zeros((L,), jnp.float32))

        npair = ((cnt + CH - 1) // CH + 1) // 2

        fire(0, rows_a, gsem_a)
        fire(1, rows_b, gsem_b)

        def pair(jj, _):
            j0 = 2 * jj
            wait(j0, rows_a, gsem_a)
            process(j0, rows_a, lsx_a, evrow_a)
            scat_fire(rows_a, lsx_a, evrow_a, ssem_a)
            wait(j0 + 1, rows_b, gsem_b)
            process(j0 + 1, rows_b, lsx_b, evrow_b)
            scat_fire(rows_b, lsx_b, evrow_b, ssem_b)
            scat_wait(rows_a, lsx_a, evrow_a, ssem_a)
            fire(j0 + 2, rows_a, gsem_a)
            scat_wait(rows_b, lsx_b, evrow_b, ssem_b)
            fire(j0 + 3, rows_b, gsem_b)
            return 0

        lax.fori_loop(0, npair, pair, 0)
        wait(2 * npair, rows_a, gsem_a)
        wait(2 * npair + 1, rows_b, gsem_b)

    plsc.subcore_barrier()

    pltpu.sync_copy(den_sh.at[pl.ds(HTS * s, HTS)], rd_v)
    for v in range(HTS // L):
        sl = pl.ds(L * v, L)
        rd_v[sl] = 1.0 / (rd_v[sl] + 1e-16)
    for b in range(HTS // 64):
        r0 = HTS * s + 64 * b
        pltpu.sync_copy(acc_sh.at[pl.ds(r0, 64)], rows_a.at[pl.ds(0, 64)])

        def nrow(r, _):
            w = plsc.load_gather(rd_v, [jnp.full((L,), 64 * b + r,
                                                 jnp.int32)])
            for cc in range(8):
                sl = pl.ds(L * cc, L)
                rows_a[r, sl] = rows_a[r, sl] * w
            return 0

        lax.fori_loop(0, 64, nrow, 0)
        pltpu.sync_copy(rows_a.at[pl.ds(0, 64)],
                        acc_hbm.at[c, pl.ds(r0, 64)])


def _make_row_pass(gather_low):
    body = functools.partial(_row_pass_body, gather_low)
    return pl.kernel(
        body,
        out_type=[jax.ShapeDtypeStruct((NC, HN, F), jnp.float32)],
        mesh=_MESH,
        compiler_params=pltpu.CompilerParams(needs_layout_passes=False),
        scratch_types=[
            pltpu.VMEM((K, 128), jnp.int32),
            pltpu.VMEM((K, 128), jnp.float32),
            pltpu.VMEM((PCAP,), jnp.int32),
            pltpu.VMEM((PCAP,), jnp.int32),
            pltpu.VMEM((PCAP,), jnp.float32),
            pltpu.VMEM((1, CH), jnp.int32),
            pltpu.VMEM((1, CH), jnp.int32),
            pltpu.VMEM((1, CH), jnp.float32),
            pltpu.VMEM((1, CH), jnp.float32),
            pltpu.VMEM((HTS,), jnp.float32),
            pltpu.VMEM((CH, F), jnp.float32),
            pltpu.VMEM((CH, F), jnp.float32),
            pltpu.VMEM_SHARED((HN, F), jnp.float32),
            pltpu.VMEM_SHARED((HN,), jnp.float32),
            pltpu.SemaphoreType.DMA,
            pltpu.SemaphoreType.DMA,
            pltpu.SemaphoreType.DMA,
            pltpu.SemaphoreType.DMA,
        ],
    )


def _pad_idx(a, fill):
    a = a.reshape(NW, EW)
    a = jnp.pad(a, ((0, 0), (0, EWP - EW)), constant_values=fill)
    return a.reshape(NW, K, 128)


def kernel(x, edge_index, W1, a1, W2, a2):
    src = edge_index[0].astype(jnp.int32)
    eidx = edge_index[1].astype(jnp.int32)
    src3 = _pad_idx(src, N)
    eidx3 = _pad_idx(eidx, 0)
    packed = src3 | (eidx3 << _IDX_BITS)

    x_pad = jnp.pad(x, ((0, NP - N), (0, 0)))
    acols = jnp.zeros((F, 8), jnp.float32)
    acols = acols.at[:, 0].set(a1).at[:, 1].set(a2[:F])
    a2b = jnp.zeros((F, 8), jnp.float32).at[:, 0].set(a2[F:])

    h, scalT = _matmul_scal(x_pad, W1, acols)

    (e1,) = _make_score_pass(False, 0, 0)(scalT, scalT, packed)
    (fpart,) = _make_row_pass(True)(h, e1, packed)

    u, uscalT = _matmul_scal(fpart.reshape(NP, F), W2, a2b, relu_input=True)

    (e2,) = _make_score_pass(True, 1, 0)(scalT, uscalT, packed)
    (opart,) = _make_row_pass(False)(u, e2, packed)

    return _elu(opart.reshape(NP, F))


# speedup vs baseline: 1.8414x; 1.8414x over previous
import functools

import jax
import jax.numpy as jnp
from jax import lax
from jax.experimental import pallas as pl
from jax.experimental.pallas import tpu as pltpu
from jax.experimental.pallas import tpu_sc as plsc

N = 10000
M = 10000
F = 128
E = 320000

NC = 2
NS = 16
NW = NC * NS
L = 16

NP = 10240
HN = NP // NC
EW = E // NW
K = 80
EWP = K * 128
TS = NP // NS
HTS = HN // NS

_BIG_NEG = -1e30
_IDX_BITS = 14
_IDX_MASK = (1 << _IDX_BITS) - 1


def _mm_scal_body(x_ref, w_ref, a_ref, h_ref, st_ref):
    xb = x_ref[...]
    hb = jnp.dot(xb, w_ref[...], preferred_element_type=jnp.float32)
    h_ref[...] = hb
    st_ref[...] = lax.dot_general(a_ref[...], hb, (((0,), (1,)), ((), ())),
                                  preferred_element_type=jnp.float32)


def _matmul_scal(x, w, acols, relu_input=False):
    blk = 1024
    body = _relu_mm_scal_body if relu_input else _mm_scal_body
    return pl.pallas_call(
        body,
        grid=(NP // blk,),
        in_specs=[
            pl.BlockSpec((blk, F), lambda i: (i, 0)),
            pl.BlockSpec((F, F), lambda i: (0, 0)),
            pl.BlockSpec((F, 8), lambda i: (0, 0)),
        ],
        out_specs=[
            pl.BlockSpec((blk, F), lambda i: (i, 0)),
            pl.BlockSpec((8, blk), lambda i: (0, i)),
        ],
        out_shape=[
            jax.ShapeDtypeStruct((NP, F), jnp.float32),
            jax.ShapeDtypeStruct((8, NP), jnp.float32),
        ],
    )(x, w, acols)


def _relu_mm_scal_body(f_ref, w_ref, a_ref, u_ref, st_ref):
    fb = jnp.maximum(f_ref[...], 0.0)
    ub = jnp.dot(fb, w_ref[...], preferred_element_type=jnp.float32)
    u_ref[...] = ub
    st_ref[...] = lax.dot_general(a_ref[...], ub, (((0,), (1,)), ((), ())),
                                  preferred_element_type=jnp.float32)


def _elu_body(o_ref, out_ref):
    o = o_ref[...]
    out_ref[...] = jnp.where(o > 0, o, jnp.exp(o) - 1.0)


def _elu(o_full):
    blk = 1000
    return pl.pallas_call(
        _elu_body,
        grid=(N // blk,),
        in_specs=[pl.BlockSpec((blk, F), lambda i: (i, 0))],
        out_specs=pl.BlockSpec((blk, F), lambda i: (i, 0)),
        out_shape=jax.ShapeDtypeStruct((N, F), jnp.float32),
    )(o_full)


_MESH = plsc.VectorSubcoreMesh(core_axis_name="c", subcore_axis_name="s",
                               num_cores=NC, num_subcores=NS)


PCAP = K * 128 + 144


def _row_pass_body(two_tables, t1_row, t2_row, gather_low,
                   rows_hbm, t1_hbm, t2_hbm, pk_hbm, acc_hbm,
                   t1_v, t2_v, pk_v, pend_g, pend_l, pend_e,
                   lsx, den_local, dtmp, rd_v, rows_a, acc_sh, den_grid):
    c = lax.axis_index("c")
    s = lax.axis_index("s")
    base = c * HN

    pltpu.sync_copy(t1_hbm.at[t1_row], t1_v)
    if two_tables:
        pltpu.sync_copy(t2_hbm.at[t2_row], t2_v)

    for t in range((NP - N) // L):
        t1_v[pl.ds(N + L * t, L)] = jnp.full((L,), _BIG_NEG, jnp.float32)

    def zrow(r, _):
        for cc in range(8):
            rows_a[r, pl.ds(L * cc, L)] = jnp.zeros((L,), jnp.float32)
        return 0

    lax.fori_loop(0, 128, zrow, 0)
    for b in range(HTS // 128):
        pltpu.sync_copy(rows_a, acc_sh.at[pl.ds(HTS * s + 128 * b, 128)])
    pltpu.sync_copy(rows_a.at[pl.ds(0, HTS % 128)],
                    acc_sh.at[pl.ds(HTS * s + (HTS // 128) * 128, HTS % 128)])
    def zden(v, _):
        den_local[pl.ds(L * v, L)] = jnp.zeros((L,), jnp.float32)
        return 0

    lax.fori_loop(0, HN // L, zden, 0)
    plsc.subcore_barrier()

    def process(j, buf):
        for cc in range(8):
            sl = pl.ds(L * cc, L)
            p = pl.ds(128 * j + L * cc, L)
            lsx[0, sl] = pend_l[p]

        def scale(r, _):
            w = plsc.load_gather(pend_e, [jnp.full((L,), 128 * j + r,
                                                   jnp.int32)])
            for cc in range(8):
                sl = pl.ds(L * cc, L)
                buf[r, sl] = buf[r, sl] * w
            return 0

        lax.fori_loop(0, 128, scale, 0)
        pltpu.sync_copy(buf, acc_sh.at[lsx.at[0]], add=True)

    for wsub in range(NC):
        wid = s * NC + wsub
        pltpu.sync_copy(pk_hbm.at[wid], pk_v)

        def compact(j, cnt):
            for cc in range(8):
                sl = pl.ds(L * cc, L)
                p16 = pk_v[j, sl]
                lo = p16 & _IDX_MASK
                hi = lax.shift_right_logical(p16, _IDX_BITS)
                if gather_low:
                    g16, s16 = lo, hi
                else:
                    g16, s16 = hi, lo
                if two_tables:
                    raw = (plsc.load_gather(t1_v, [s16])
                           + plsc.load_gather(t2_v, [g16]))
                else:
                    raw = plsc.load_gather(t1_v, [g16])
                raw = jnp.where(raw > 0, raw, 0.2 * raw)
                e16 = jnp.exp(raw)
                l16 = s16 - base
                msk = (l16 >= 0) & (l16 < HN)
                mi = msk.astype(jnp.int32)
                pos = cnt + plsc.cumsum(mi) - 1
                plsc.store_scatter(pend_g, [pos], g16, mask=msk)
                plsc.store_scatter(pend_l, [pos], l16, mask=msk)
                plsc.store_scatter(pend_e, [pos], e16, mask=msk)
                plsc.addupdate_scatter(den_local, [l16], e16, mask=msk)
                cnt = cnt + jnp.sum(mi)
            return cnt

        cnt = lax.fori_loop(0, K, compact, jnp.int32(0))

        iota16 = lax.iota(jnp.int32, L)
        for t in range(8):
            pos = cnt + iota16 + L * t
            plsc.store_scatter(pend_g, [pos], jnp.zeros((L,), jnp.int32))
            plsc.store_scatter(pend_l, [pos], jnp.zeros((L,), jnp.int32))
            plsc.store_scatter(pend_e, [pos], jnp.zeros((L,), jnp.float32))

        nch = (cnt + 127) // 128

        def chunk(j, _):
            pltpu.sync_copy(rows_hbm.at[pend_g.at[pl.ds(128 * j, 128)]],
                            rows_a)
            process(j, rows_a)
            return 0

        lax.fori_loop(0, nch, chunk, 0)

    pltpu.sync_copy(den_local, den_grid.at[pl.ds(HN * s, HN)])
    plsc.subcore_barrier()

    pltpu.sync_copy(den_grid.at[pl.ds(HTS * s, HTS)], rd_v)
    for t in range(1, NS):
        pltpu.sync_copy(den_grid.at[pl.ds(HN * t + HTS * s, HTS)], dtmp)
        for v in range(HTS // L):
            sl = pl.ds(L * v, L)
            rd_v[sl] = rd_v[sl] + dtmp[sl]
    for v in range(HTS // L):
        sl = pl.ds(L * v, L)
        rd_v[sl] = 1.0 / (rd_v[sl] + 1e-16)
    for b in range(HTS // 64):
        r0 = HTS * s + 64 * b
        pltpu.sync_copy(acc_sh.at[pl.ds(r0, 64)], rows_a.at[pl.ds(0, 64)])

        def nrow(r, _):
            w = plsc.load_gather(rd_v, [jnp.full((L,), 64 * b + r,
                                                 jnp.int32)])
            for cc in range(8):
                sl = pl.ds(L * cc, L)
                rows_a[r, sl] = rows_a[r, sl] * w
            return 0

        lax.fori_loop(0, 64, nrow, 0)
        pltpu.sync_copy(rows_a.at[pl.ds(0, 64)],
                        acc_hbm.at[c, pl.ds(r0, 64)])


def _make_row_pass(two_tables, t1_row, t2_row, gather_low):
    body = functools.partial(_row_pass_body, two_tables, t1_row, t2_row,
                             gather_low)
    return pl.kernel(
        body,
        out_type=[jax.ShapeDtypeStruct((NC, HN, F), jnp.float32)],
        mesh=_MESH,
        compiler_params=pltpu.CompilerParams(needs_layout_passes=False),
        scratch_types=[
            pltpu.VMEM((NP,), jnp.float32),
            pltpu.VMEM((NP,), jnp.float32),
            pltpu.VMEM((K, 128), jnp.int32),
            pltpu.VMEM((PCAP,), jnp.int32),
            pltpu.VMEM((PCAP,), jnp.int32),
            pltpu.VMEM((PCAP,), jnp.float32),
            pltpu.VMEM((1, 128), jnp.int32),
            pltpu.VMEM((HN,), jnp.float32),
            pltpu.VMEM((HTS,), jnp.float32),
            pltpu.VMEM((HTS,), jnp.float32),
            pltpu.VMEM((128, F), jnp.float32),
            pltpu.VMEM_SHARED((HN, F), jnp.float32),
            pltpu.VMEM_SHARED((NS * HN,), jnp.float32),
        ],
    )


def _pad_idx(a, fill):
    a = a.reshape(NW, EW)
    a = jnp.pad(a, ((0, 0), (0, EWP - EW)), constant_values=fill)
    return a.reshape(NW, K, 128)


def kernel(x, edge_index, W1, a1, W2, a2):
    src = edge_index[0].astype(jnp.int32)
    eidx = edge_index[1].astype(jnp.int32)
    src3 = _pad_idx(src, N)
    eidx3 = _pad_idx(eidx, 0)
    packed = src3 | (eidx3 << _IDX_BITS)

    x_pad = jnp.pad(x, ((0, NP - N), (0, 0)))
    acols = jnp.zeros((F, 8), jnp.float32)
    acols = acols.at[:, 0].set(a1).at[:, 1].set(a2[:F])
    a2b = jnp.zeros((F, 8), jnp.float32).at[:, 0].set(a2[F:])

    h, scalT = _matmul_scal(x_pad, W1, acols)

    (fpart,) = _make_row_pass(False, 0, 0, True)(h, scalT, scalT, packed)

    u, uscalT = _matmul_scal(fpart.reshape(NP, F), W2, a2b, relu_input=True)

    (opart,) = _make_row_pass(True, 1, 0, False)(u, scalT, uscalT, packed)

    return _elu(opart.reshape(NP, F))


# speedup vs baseline: 1.9229x; 1.0443x over previous
import functools

import jax
import jax.numpy as jnp
from jax import lax
from jax.experimental import pallas as pl
from jax.experimental.pallas import tpu as pltpu
from jax.experimental.pallas import tpu_sc as plsc

N = 10000
M = 10000
F = 128
E = 320000

NC = 2
NS = 16
NW = NC * NS
L = 16

NP = 10240
HN = NP // NC
EW = E // NW
K = 80
EWP = K * 128
TS = NP // NS
HTS = HN // NS

_BIG_NEG = -1e30
_IDX_BITS = 14
_IDX_MASK = (1 << _IDX_BITS) - 1


def _mm_scal_body(x_ref, w_ref, a_ref, h_ref, st_ref):
    xb = x_ref[...]
    hb = jnp.dot(xb, w_ref[...], preferred_element_type=jnp.float32)
    h_ref[...] = hb
    st_ref[...] = lax.dot_general(a_ref[...], hb, (((0,), (1,)), ((), ())),
                                  preferred_element_type=jnp.float32)


def _matmul_scal(x, w, acols, relu_input=False):
    blk = 1024
    body = _relu_mm_scal_body if relu_input else _mm_scal_body
    return pl.pallas_call(
        body,
        grid=(NP // blk,),
        in_specs=[
            pl.BlockSpec((blk, F), lambda i: (i, 0)),
            pl.BlockSpec((F, F), lambda i: (0, 0)),
            pl.BlockSpec((F, 8), lambda i: (0, 0)),
        ],
        out_specs=[
            pl.BlockSpec((blk, F), lambda i: (i, 0)),
            pl.BlockSpec((8, blk), lambda i: (0, i)),
        ],
        out_shape=[
            jax.ShapeDtypeStruct((NP, F), jnp.float32),
            jax.ShapeDtypeStruct((8, NP), jnp.float32),
        ],
    )(x, w, acols)


def _relu_mm_scal_body(f_ref, w_ref, a_ref, u_ref, st_ref):
    fb = jnp.maximum(f_ref[...], 0.0)
    ub = jnp.dot(fb, w_ref[...], preferred_element_type=jnp.float32)
    u_ref[...] = ub
    st_ref[...] = lax.dot_general(a_ref[...], ub, (((0,), (1,)), ((), ())),
                                  preferred_element_type=jnp.float32)


def _elu_body(o_ref, out_ref):
    o = o_ref[...]
    out_ref[...] = jnp.where(o > 0, o, jnp.exp(o) - 1.0)


def _elu(o_full):
    blk = 1000
    return pl.pallas_call(
        _elu_body,
        grid=(N // blk,),
        in_specs=[pl.BlockSpec((blk, F), lambda i: (i, 0))],
        out_specs=pl.BlockSpec((blk, F), lambda i: (i, 0)),
        out_shape=jax.ShapeDtypeStruct((N, F), jnp.float32),
    )(o_full)


_MESH = plsc.VectorSubcoreMesh(core_axis_name="c", subcore_axis_name="s",
                               num_cores=NC, num_subcores=NS)


PCAP = K * 128 + 144


def _row_pass_body(two_tables, t1_row, t2_row, gather_low,
                   rows_hbm, t1_hbm, t2_hbm, pk_hbm, acc_hbm,
                   t1_v, t2_v, pk_v, pend_g, pend_l, pend_e,
                   lsx, den_local, dtmp, rd_v, rows_a, acc_sh, den_grid):
    c = lax.axis_index("c")
    s = lax.axis_index("s")
    base = c * HN

    pltpu.sync_copy(t1_hbm.at[t1_row], t1_v)
    if two_tables:
        pltpu.sync_copy(t2_hbm.at[t2_row], t2_v)

    for t in range((NP - N) // L):
        t1_v[pl.ds(N + L * t, L)] = jnp.full((L,), _BIG_NEG, jnp.float32)

    def zrow(r, _):
        for cc in range(8):
            rows_a[r, pl.ds(L * cc, L)] = jnp.zeros((L,), jnp.float32)
        return 0

    lax.fori_loop(0, 128, zrow, 0)
    for b in range(HTS // 128):
        pltpu.sync_copy(rows_a, acc_sh.at[pl.ds(HTS * s + 128 * b, 128)])
    pltpu.sync_copy(rows_a.at[pl.ds(0, HTS % 128)],
                    acc_sh.at[pl.ds(HTS * s + (HTS // 128) * 128, HTS % 128)])
    def zden(v, _):
        den_local[pl.ds(L * v, L)] = jnp.zeros((L,), jnp.float32)
        return 0

    lax.fori_loop(0, HN // L, zden, 0)
    plsc.subcore_barrier()

    def process(j, buf):
        for cc in range(8):
            sl = pl.ds(L * cc, L)
            p = pl.ds(128 * j + L * cc, L)
            lsx[0, sl] = pend_l[p]

        def scale(r2, _):
            r = 2 * r2
            w0 = plsc.load_gather(pend_e, [jnp.full((L,), 128 * j + r,
                                                    jnp.int32)])
            w1 = plsc.load_gather(pend_e, [jnp.full((L,), 128 * j + r + 1,
                                                    jnp.int32)])
            for cc in range(8):
                sl = pl.ds(L * cc, L)
                buf[r, sl] = buf[r, sl] * w0
            for cc in range(8):
                sl = pl.ds(L * cc, L)
                buf[r + 1, sl] = buf[r + 1, sl] * w1
            return 0

        lax.fori_loop(0, 64, scale, 0)
        pltpu.sync_copy(buf, acc_sh.at[lsx.at[0]], add=True)

    for wsub in range(NC):
        wid = s * NC + wsub
        pltpu.sync_copy(pk_hbm.at[wid], pk_v)

        def compact(j, cnt):
            for cc in range(8):
                sl = pl.ds(L * cc, L)
                p16 = pk_v[j, sl]
                lo = p16 & _IDX_MASK
                hi = lax.shift_right_logical(p16, _IDX_BITS)
                if gather_low:
                    g16, s16 = lo, hi
                else:
                    g16, s16 = hi, lo
                if two_tables:
                    raw = (plsc.load_gather(t1_v, [s16])
                           + plsc.load_gather(t2_v, [g16]))
                else:
                    raw = plsc.load_gather(t1_v, [g16])
                raw = jnp.where(raw > 0, raw, 0.2 * raw)
                e16 = jnp.exp(raw)
                l16 = s16 - base
                msk = (l16 >= 0) & (l16 < HN)
                mi = msk.astype(jnp.int32)
                pos = cnt + plsc.cumsum(mi) - 1
                plsc.store_scatter(pend_g, [pos], g16, mask=msk)
                plsc.store_scatter(pend_l, [pos], l16, mask=msk)
                plsc.store_scatter(pend_e, [pos], e16, mask=msk)
                plsc.addupdate_scatter(den_local, [l16], e16, mask=msk)
                cnt = cnt + jnp.sum(mi)
            return cnt

        cnt = lax.fori_loop(0, K, compact, jnp.int32(0))

        iota16 = lax.iota(jnp.int32, L)
        for t in range(8):
            pos = cnt + iota16 + L * t
            plsc.store_scatter(pend_g, [pos], jnp.zeros((L,), jnp.int32))
            plsc.store_scatter(pend_l, [pos], jnp.zeros((L,), jnp.int32))
            plsc.store_scatter(pend_e, [pos], jnp.zeros((L,), jnp.float32))

        nch = (cnt + 127) // 128

        def chunk(j, _):
            pltpu.sync_copy(rows_hbm.at[pend_g.at[pl.ds(128 * j, 128)]],
                            rows_a)
            process(j, rows_a)
            return 0

        lax.fori_loop(0, nch, chunk, 0)

    pltpu.sync_copy(den_local, den_grid.at[pl.ds(HN * s, HN)])
    plsc.subcore_barrier()

    pltpu.sync_copy(den_grid.at[pl.ds(HTS * s, HTS)], rd_v)
    for t in range(1, NS):
        pltpu.sync_copy(den_grid.at[pl.ds(HN * t + HTS * s, HTS)], dtmp)
        for v in range(HTS // L):
            sl = pl.ds(L * v, L)
            rd_v[sl] = rd_v[sl] + dtmp[sl]
    for v in range(HTS // L):
        sl = pl.ds(L * v, L)
        rd_v[sl] = 1.0 / (rd_v[sl] + 1e-16)
    for b in range(HTS // 64):
        r0 = HTS * s + 64 * b
        pltpu.sync_copy(acc_sh.at[pl.ds(r0, 64)], rows_a.at[pl.ds(0, 64)])

        def nrow(r2, _):
            r = 2 * r2
            w0 = plsc.load_gather(rd_v, [jnp.full((L,), 64 * b + r,
                                                  jnp.int32)])
            w1 = plsc.load_gather(rd_v, [jnp.full((L,), 64 * b + r + 1,
                                                  jnp.int32)])
            for cc in range(8):
                sl = pl.ds(L * cc, L)
                rows_a[r, sl] = rows_a[r, sl] * w0
            for cc in range(8):
                sl = pl.ds(L * cc, L)
                rows_a[r + 1, sl] = rows_a[r + 1, sl] * w1
            return 0

        lax.fori_loop(0, 32, nrow, 0)
        pltpu.sync_copy(rows_a.at[pl.ds(0, 64)],
                        acc_hbm.at[c, pl.ds(r0, 64)])


def _make_row_pass(two_tables, t1_row, t2_row, gather_low):
    body = functools.partial(_row_pass_body, two_tables, t1_row, t2_row,
                             gather_low)
    return pl.kernel(
        body,
        out_type=[jax.ShapeDtypeStruct((NC, HN, F), jnp.float32)],
        mesh=_MESH,
        compiler_params=pltpu.CompilerParams(needs_layout_passes=False),
        scratch_types=[
            pltpu.VMEM((NP,), jnp.float32),
            pltpu.VMEM((NP,), jnp.float32),
            pltpu.VMEM((K, 128), jnp.int32),
            pltpu.VMEM((PCAP,), jnp.int32),
            pltpu.VMEM((PCAP,), jnp.int32),
            pltpu.VMEM((PCAP,), jnp.float32),
            pltpu.VMEM((1, 128), jnp.int32),
            pltpu.VMEM((HN,), jnp.float32),
            pltpu.VMEM((HTS,), jnp.float32),
            pltpu.VMEM((HTS,), jnp.float32),
            pltpu.VMEM((128, F), jnp.float32),
            pltpu.VMEM_SHARED((HN, F), jnp.float32),
            pltpu.VMEM_SHARED((NS * HN,), jnp.float32),
        ],
    )


def _pad_idx(a, fill):
    a = a.reshape(NW, EW)
    a = jnp.pad(a, ((0, 0), (0, EWP - EW)), constant_values=fill)
    return a.reshape(NW, K, 128)


def kernel(x, edge_index, W1, a1, W2, a2):
    src = edge_index[0].astype(jnp.int32)
    eidx = edge_index[1].astype(jnp.int32)
    src3 = _pad_idx(src, N)
    eidx3 = _pad_idx(eidx, 0)
    packed = src3 | (eidx3 << _IDX_BITS)

    x_pad = jnp.pad(x, ((0, NP - N), (0, 0)))
    acols = jnp.zeros((F, 8), jnp.float32)
    acols = acols.at[:, 0].set(a1).at[:, 1].set(a2[:F])
    a2b = jnp.zeros((F, 8), jnp.float32).at[:, 0].set(a2[F:])

    h, scalT = _matmul_scal(x_pad, W1, acols)

    (fpart,) = _make_row_pass(False, 0, 0, True)(h, scalT, scalT, packed)

    u, uscalT = _matmul_scal(fpart.reshape(NP, F), W2, a2b, relu_input=True)

    (opart,) = _make_row_pass(True, 1, 0, False)(u, scalT, uscalT, packed)

    return _elu(opart.reshape(NP, F))


# speedup vs baseline: 1.9584x; 1.0185x over previous
import functools

import jax
import jax.numpy as jnp
from jax import lax
from jax.experimental import pallas as pl
from jax.experimental.pallas import tpu as pltpu
from jax.experimental.pallas import tpu_sc as plsc

N = 10000
M = 10000
F = 128
E = 320000

NC = 2
NS = 16
NW = NC * NS
L = 16

NP = 10240
HN = NP // NC
EW = E // NW
K = 80
EWP = K * 128
TS = NP // NS
HTS = HN // NS

_BIG_NEG = -1e30
_IDX_BITS = 14
_IDX_MASK = (1 << _IDX_BITS) - 1


def _mm_scal_body(x_ref, w_ref, a_ref, h_ref, st_ref):
    xb = x_ref[...]
    hb = jnp.dot(xb, w_ref[...], preferred_element_type=jnp.float32)
    h_ref[...] = hb
    st_ref[...] = lax.dot_general(a_ref[...], hb, (((0,), (1,)), ((), ())),
                                  preferred_element_type=jnp.float32)


def _matmul_scal(x, w, acols, relu_input=False):
    blk = 1024
    body = _relu_mm_scal_body if relu_input else _mm_scal_body
    return pl.pallas_call(
        body,
        grid=(NP // blk,),
        in_specs=[
            pl.BlockSpec((blk, F), lambda i: (i, 0)),
            pl.BlockSpec((F, F), lambda i: (0, 0)),
            pl.BlockSpec((F, 8), lambda i: (0, 0)),
        ],
        out_specs=[
            pl.BlockSpec((blk, F), lambda i: (i, 0)),
            pl.BlockSpec((8, blk), lambda i: (0, i)),
        ],
        out_shape=[
            jax.ShapeDtypeStruct((NP, F), jnp.float32),
            jax.ShapeDtypeStruct((8, NP), jnp.float32),
        ],
    )(x, w, acols)


def _relu_mm_scal_body(f_ref, w_ref, a_ref, u_ref, st_ref):
    fb = jnp.maximum(f_ref[...], 0.0)
    ub = jnp.dot(fb, w_ref[...], preferred_element_type=jnp.float32)
    u_ref[...] = ub
    st_ref[...] = lax.dot_general(a_ref[...], ub, (((0,), (1,)), ((), ())),
                                  preferred_element_type=jnp.float32)


def _elu_body(o_ref, out_ref):
    o = o_ref[...]
    out_ref[...] = jnp.where(o > 0, o, jnp.exp(o) - 1.0)


def _elu(o_full):
    blk = 1000
    return pl.pallas_call(
        _elu_body,
        grid=(N // blk,),
        in_specs=[pl.BlockSpec((blk, F), lambda i: (i, 0))],
        out_specs=pl.BlockSpec((blk, F), lambda i: (i, 0)),
        out_shape=jax.ShapeDtypeStruct((N, F), jnp.float32),
    )(o_full)


_MESH = plsc.VectorSubcoreMesh(core_axis_name="c", subcore_axis_name="s",
                               num_cores=NC, num_subcores=NS)


PCAP = K * 128 + 144


def _row_pass_body(two_tables, t1_row, t2_row, gather_low,
                   rows_hbm, t1_hbm, t2_hbm, pk_hbm, acc_hbm,
                   t1_v, t2_v, pk_v, pend_g, pend_l, pend_e,
                   lsx, den_local, dtmp, rd_v, rows_a, acc_sh, den_grid):
    c = lax.axis_index("c")
    s = lax.axis_index("s")
    base = c * HN

    pltpu.sync_copy(t1_hbm.at[t1_row], t1_v)
    if two_tables:
        pltpu.sync_copy(t2_hbm.at[t2_row], t2_v)

    for t in range((NP - N) // L):
        t1_v[pl.ds(N + L * t, L)] = jnp.full((L,), _BIG_NEG, jnp.float32)

    def zrow(r, _):
        for cc in range(8):
            rows_a[r, pl.ds(L * cc, L)] = jnp.zeros((L,), jnp.float32)
        return 0

    lax.fori_loop(0, 128, zrow, 0)
    for b in range(HTS // 128):
        pltpu.sync_copy(rows_a, acc_sh.at[pl.ds(HTS * s + 128 * b, 128)])
    pltpu.sync_copy(rows_a.at[pl.ds(0, HTS % 128)],
                    acc_sh.at[pl.ds(HTS * s + (HTS // 128) * 128, HTS % 128)])
    def zden(v, _):
        den_local[pl.ds(L * v, L)] = jnp.zeros((L,), jnp.float32)
        return 0

    lax.fori_loop(0, HN // L, zden, 0)
    plsc.subcore_barrier()

    def process(j, buf):
        for cc in range(8):
            sl = pl.ds(L * cc, L)
            p = pl.ds(128 * j + L * cc, L)
            lsx[0, sl] = pend_l[p]

        def scale(r4, _):
            r = 4 * r4
            ws = [plsc.load_gather(pend_e, [jnp.full((L,), 128 * j + r + u,
                                                     jnp.int32)])
                  for u in range(4)]
            for u in range(4):
                for cc in range(8):
                    sl = pl.ds(L * cc, L)
                    buf[r + u, sl] = buf[r + u, sl] * ws[u]
            return 0

        lax.fori_loop(0, 32, scale, 0)
        pltpu.sync_copy(buf, acc_sh.at[lsx.at[0]], add=True)

    for wsub in range(NC):
        wid = s * NC + wsub
        pltpu.sync_copy(pk_hbm.at[wid], pk_v)

        def compact(j, cnt):
            for cc in range(8):
                sl = pl.ds(L * cc, L)
                p16 = pk_v[j, sl]
                lo = p16 & _IDX_MASK
                hi = lax.shift_right_logical(p16, _IDX_BITS)
                if gather_low:
                    g16, s16 = lo, hi
                else:
                    g16, s16 = hi, lo
                if two_tables:
                    raw = (plsc.load_gather(t1_v, [s16])
                           + plsc.load_gather(t2_v, [g16]))
                else:
                    raw = plsc.load_gather(t1_v, [g16])
                raw = jnp.where(raw > 0, raw, 0.2 * raw)
                e16 = jnp.exp(raw)
                l16 = s16 - base
                msk = (l16 >= 0) & (l16 < HN)
                mi = msk.astype(jnp.int32)
                pos = cnt + plsc.cumsum(mi) - 1
                plsc.store_scatter(pend_g, [pos], g16, mask=msk)
                plsc.store_scatter(pend_l, [pos], l16, mask=msk)
                plsc.store_scatter(pend_e, [pos], e16, mask=msk)
                plsc.addupdate_scatter(den_local, [l16], e16, mask=msk)
                cnt = cnt + jnp.sum(mi)
            return cnt

        cnt = lax.fori_loop(0, K, compact, jnp.int32(0))

        iota16 = lax.iota(jnp.int32, L)
        for t in range(8):
            pos = cnt + iota16 + L * t
            plsc.store_scatter(pend_g, [pos], jnp.zeros((L,), jnp.int32))
            plsc.store_scatter(pend_l, [pos], jnp.zeros((L,), jnp.int32))
            plsc.store_scatter(pend_e, [pos], jnp.zeros((L,), jnp.float32))

        nch = (cnt + 127) // 128

        def chunk(j, _):
            pltpu.sync_copy(rows_hbm.at[pend_g.at[pl.ds(128 * j, 128)]],
                            rows_a)
            process(j, rows_a)
            return 0

        lax.fori_loop(0, nch, chunk, 0)

    pltpu.sync_copy(den_local, den_grid.at[pl.ds(HN * s, HN)])
    plsc.subcore_barrier()

    pltpu.sync_copy(den_grid.at[pl.ds(HTS * s, HTS)], rd_v)
    for t in range(1, NS):
        pltpu.sync_copy(den_grid.at[pl.ds(HN * t + HTS * s, HTS)], dtmp)
        for v in range(HTS // L):
            sl = pl.ds(L * v, L)
            rd_v[sl] = rd_v[sl] + dtmp[sl]
    for v in range(HTS // L):
        sl = pl.ds(L * v, L)
        rd_v[sl] = 1.0 / (rd_v[sl] + 1e-16)
    for b in range(HTS // 64):
        r0 = HTS * s + 64 * b
        pltpu.sync_copy(acc_sh.at[pl.ds(r0, 64)], rows_a.at[pl.ds(0, 64)])

        def nrow(r2, _):
            r = 2 * r2
            w0 = plsc.load_gather(rd_v, [jnp.full((L,), 64 * b + r,
                                                  jnp.int32)])
            w1 = plsc.load_gather(rd_v, [jnp.full((L,), 64 * b + r + 1,
                                                  jnp.int32)])
            for cc in range(8):
                sl = pl.ds(L * cc, L)
                rows_a[r, sl] = rows_a[r, sl] * w0
            for cc in range(8):
                sl = pl.ds(L * cc, L)
                rows_a[r + 1, sl] = rows_a[r + 1, sl] * w1
            return 0

        lax.fori_loop(0, 32, nrow, 0)
        pltpu.sync_copy(rows_a.at[pl.ds(0, 64)],
                        acc_hbm.at[c, pl.ds(r0, 64)])


def _make_row_pass(two_tables, t1_row, t2_row, gather_low):
    body = functools.partial(_row_pass_body, two_tables, t1_row, t2_row,
                             gather_low)
    return pl.kernel(
        body,
        out_type=[jax.ShapeDtypeStruct((NC, HN, F), jnp.float32)],
        mesh=_MESH,
        compiler_params=pltpu.CompilerParams(needs_layout_passes=False),
        scratch_types=[
            pltpu.VMEM((NP,), jnp.float32),
            pltpu.VMEM((NP,), jnp.float32),
            pltpu.VMEM((K, 128), jnp.int32),
            pltpu.VMEM((PCAP,), jnp.int32),
            pltpu.VMEM((PCAP,), jnp.int32),
            pltpu.VMEM((PCAP,), jnp.float32),
            pltpu.VMEM((1, 128), jnp.int32),
            pltpu.VMEM((HN,), jnp.float32),
            pltpu.VMEM((HTS,), jnp.float32),
            pltpu.VMEM((HTS,), jnp.float32),
            pltpu.VMEM((128, F), jnp.float32),
            pltpu.VMEM_SHARED((HN, F), jnp.float32),
            pltpu.VMEM_SHARED((NS * HN,), jnp.float32),
        ],
    )


def _pad_idx(a, fill):
    a = a.reshape(NW, EW)
    a = jnp.pad(a, ((0, 0), (0, EWP - EW)), constant_values=fill)
    return a.reshape(NW, K, 128)


def kernel(x, edge_index, W1, a1, W2, a2):
    src = edge_index[0].astype(jnp.int32)
    eidx = edge_index[1].astype(jnp.int32)
    src3 = _pad_idx(src, N)
    eidx3 = _pad_idx(eidx, 0)
    packed = src3 | (eidx3 << _IDX_BITS)

    x_pad = jnp.pad(x, ((0, NP - N), (0, 0)))
    acols = jnp.zeros((F, 8), jnp.float32)
    acols = acols.at[:, 0].set(a1).at[:, 1].set(a2[:F])
    a2b = jnp.zeros((F, 8), jnp.float32).at[:, 0].set(a2[F:])

    h, scalT = _matmul_scal(x_pad, W1, acols)

    (fpart,) = _make_row_pass(False, 0, 0, True)(h, scalT, scalT, packed)

    u, uscalT = _matmul_scal(fpart.reshape(NP, F), W2, a2b, relu_input=True)

    (opart,) = _make_row_pass(True, 1, 0, False)(u, scalT, uscalT, packed)

    return _elu(opart.reshape(NP, F))
